# Initial kernel scaffold; baseline (speedup 1.0000x reference)
import jax
import jax.numpy as jnp
from jax.experimental import pallas as pl


def kernel(x, offset, grid_size):
    raise NotImplementedError("write your pallas kernel here")


# speedup vs baseline: 5.1244x; 5.1244x over previous
import functools

import jax
import jax.numpy as jnp
from jax import lax
from jax.experimental import pallas as pl
from jax.experimental.pallas import tpu as pltpu
from jax.experimental.pallas import tpu_sc as plsc

B, C, H, W = 8, 192, 112, 112
GH, GW = 56, 56
P = GH * GW
HW = H * W
NPLANES = B * C
L = 16
NG = P // L
NWORK = 32
WPB = NWORK // B
CPW = C // WPB

SY = float(H) / GH
SX = float(W) / GW
BY = SY * 0.5 - 0.5
BX = SX * 0.5 - 0.5


def _ifloor(v):
    t = v.astype(jnp.int32)
    return jnp.where(t.astype(jnp.float32) > v, t - 1, t)


_mesh = plsc.VectorSubcoreMesh(core_axis_name="c", subcore_axis_name="s")


@functools.partial(
    pl.kernel,
    mesh=_mesh,
    compiler_params=pltpu.CompilerParams(needs_layout_passes=False),
    out_type=jax.ShapeDtypeStruct((NPLANES, P), jnp.float32),
    scratch_types=[
        pltpu.VMEM((2 * P,), jnp.float32),
        pltpu.VMEM((P,), jnp.int32),
        pltpu.VMEM((P,), jnp.int32),
        pltpu.VMEM((P,), jnp.int32),
        pltpu.VMEM((P,), jnp.int32),
        pltpu.VMEM((P,), jnp.float32),
        pltpu.VMEM((P,), jnp.float32),
        pltpu.VMEM((P,), jnp.float32),
        pltpu.VMEM((P,), jnp.float32),
        pltpu.VMEM((HW,), jnp.float32),
        pltpu.VMEM((HW,), jnp.float32),
        pltpu.VMEM((P,), jnp.float32),
        pltpu.SemaphoreType.DMA,
        pltpu.SemaphoreType.DMA,
    ],
)
def _defem_sc(x_hbm, off_hbm, out_hbm,
              off_v, i00, i01, i10, i11, w00, w01, w10, w11,
              plane0, plane1, outbuf, sem0, sem1):
    cid = lax.axis_index("c")
    sid = lax.axis_index("s")
    wid = sid * 2 + cid
    b = lax.div(wid, jnp.int32(WPB))
    cbase = lax.rem(wid, jnp.int32(WPB)) * CPW

    pltpu.sync_copy(off_hbm.at[b], off_v)

    def ibody(g, carry):
        s = g * L
        lane = lax.iota(jnp.int32, L)
        p = s + lane
        pi = lax.div(p, jnp.int32(GW))
        pj = p - pi * GW
        ys = pi.astype(jnp.float32) * SY + BY + off_v[pl.ds(s, L)]
        xs = pj.astype(jnp.float32) * SX + BX + off_v[pl.ds(P + s, L)]
        y0 = _ifloor(ys)
        x0 = _ifloor(xs)
        fy1 = ys - y0.astype(jnp.float32)
        fy0 = 1.0 - fy1
        fx1 = xs - x0.astype(jnp.float32)
        fx0 = 1.0 - fx1
        wy0 = jnp.where((y0 >= 0) & (y0 <= H - 1), fy0, 0.0)
        wy1 = jnp.where((y0 >= -1) & (y0 <= H - 2), fy1, 0.0)
        wx0 = jnp.where((x0 >= 0) & (x0 <= W - 1), fx0, 0.0)
        wx1 = jnp.where((x0 >= -1) & (x0 <= W - 2), fx1, 0.0)
        yc0 = jnp.clip(y0, 0, H - 1)
        yc1 = jnp.clip(y0 + 1, 0, H - 1)
        xc0 = jnp.clip(x0, 0, W - 1)
        xc1 = jnp.clip(x0 + 1, 0, W - 1)
        r0 = yc0 * W
        r1 = yc1 * W
        i00[pl.ds(s, L)] = r0 + xc0
        i01[pl.ds(s, L)] = r0 + xc1
        i10[pl.ds(s, L)] = r1 + xc0
        i11[pl.ds(s, L)] = r1 + xc1
        w00[pl.ds(s, L)] = wy0 * wx0
        w01[pl.ds(s, L)] = wy0 * wx1
        w10[pl.ds(s, L)] = wy1 * wx0
        w11[pl.ds(s, L)] = wy1 * wx1
        return carry

    lax.fori_loop(0, NG, ibody, None)

    planes = (plane0, plane1)
    sems = (sem0, sem1)
    row0 = b * C + cbase

    def compute_plane(plane):
        def gbody(g, carry):
            s = g * L
            acc = plsc.load_gather(plane, [i00[pl.ds(s, L)]]) * w00[pl.ds(s, L)]
            acc = acc + plsc.load_gather(plane, [i01[pl.ds(s, L)]]) * w01[pl.ds(s, L)]
            acc = acc + plsc.load_gather(plane, [i10[pl.ds(s, L)]]) * w10[pl.ds(s, L)]
            acc = acc + plsc.load_gather(plane, [i11[pl.ds(s, L)]]) * w11[pl.ds(s, L)]
            outbuf[pl.ds(s, L)] = acc
            return carry

        lax.fori_loop(0, NG, gbody, None)

    handles = [None, None]
    handles[0] = pltpu.async_copy(x_hbm.at[row0], plane0, sem0)
    for k in range(CPW):
        cur = k % 2
        nxt = 1 - cur
        if k + 1 < CPW:
            handles[nxt] = pltpu.async_copy(
                x_hbm.at[row0 + (k + 1)], planes[nxt], sems[nxt])
        handles[cur].wait()
        compute_plane(planes[cur])
        pltpu.sync_copy(outbuf, out_hbm.at[row0 + k])


def kernel(x, offset, grid_size):
    shift = jnp.asarray(grid_size).astype(jnp.float32) - jnp.float32(GH)
    off = offset.reshape(B, 2 * P) + shift
    out = _defem_sc(x.reshape(NPLANES, HW), off)
    return out.reshape(B, C, GH, GW)


# speedup vs baseline: 5.3494x; 1.0439x over previous
import functools

import jax
import jax.numpy as jnp
from jax import lax
from jax.experimental import pallas as pl
from jax.experimental.pallas import tpu as pltpu
from jax.experimental.pallas import tpu_sc as plsc

B, C, H, W = 8, 192, 112, 112
GH, GW = 56, 56
P = GH * GW
HW = H * W
NPLANES = B * C
L = 16
NG = P // L
NWORK = 32
WPB = NWORK // B
CPW = C // WPB

SY = float(H) / GH
SX = float(W) / GW
BY = SY * 0.5 - 0.5
BX = SX * 0.5 - 0.5

NCH = 3
NBLK = CPW // NCH


def _ifloor(v):
    t = v.astype(jnp.int32)
    return jnp.where(t.astype(jnp.float32) > v, t - 1, t)


_mesh = plsc.VectorSubcoreMesh(core_axis_name="c", subcore_axis_name="s")


@functools.partial(
    pl.kernel,
    mesh=_mesh,
    compiler_params=pltpu.CompilerParams(needs_layout_passes=False),
    out_type=jax.ShapeDtypeStruct((NPLANES, P), jnp.float32),
    scratch_types=[
        pltpu.VMEM((2 * P,), jnp.float32),
        pltpu.VMEM((P,), jnp.int32),
        pltpu.VMEM((P,), jnp.int32),
        pltpu.VMEM((P,), jnp.int32),
        pltpu.VMEM((P,), jnp.int32),
        pltpu.VMEM((P,), jnp.float32),
        pltpu.VMEM((P,), jnp.float32),
        pltpu.VMEM((P,), jnp.float32),
        pltpu.VMEM((P,), jnp.float32),
        pltpu.VMEM((HW,), jnp.float32),
        pltpu.VMEM((HW,), jnp.float32),
        pltpu.VMEM((HW,), jnp.float32),
        pltpu.VMEM((HW,), jnp.float32),
        pltpu.VMEM((HW,), jnp.float32),
        pltpu.VMEM((HW,), jnp.float32),
        pltpu.VMEM((P,), jnp.float32),
        pltpu.VMEM((P,), jnp.float32),
        pltpu.VMEM((P,), jnp.float32),
        pltpu.SemaphoreType.DMA,
        pltpu.SemaphoreType.DMA,
        pltpu.SemaphoreType.DMA,
        pltpu.SemaphoreType.DMA,
        pltpu.SemaphoreType.DMA,
        pltpu.SemaphoreType.DMA,
    ],
)
def _defem_sc(x_hbm, off_hbm, out_hbm,
              off_v, i00, i01, i10, i11, w00, w01, w10, w11,
              pA0, pA1, pA2, pB0, pB1, pB2, ob0, ob1, ob2,
              sA0, sA1, sA2, sB0, sB1, sB2):
    cid = lax.axis_index("c")
    sid = lax.axis_index("s")
    wid = sid * 2 + cid
    b = lax.div(wid, jnp.int32(WPB))
    cbase = lax.rem(wid, jnp.int32(WPB)) * CPW

    pltpu.sync_copy(off_hbm.at[b], off_v)

    def ibody(g, carry):
        s = g * L
        lane = lax.iota(jnp.int32, L)
        p = s + lane
        pi = lax.div(p, jnp.int32(GW))
        pj = p - pi * GW
        ys = pi.astype(jnp.float32) * SY + BY + off_v[pl.ds(s, L)]
        xs = pj.astype(jnp.float32) * SX + BX + off_v[pl.ds(P + s, L)]
        y0 = _ifloor(ys)
        x0 = _ifloor(xs)
        fy1 = ys - y0.astype(jnp.float32)
        fy0 = 1.0 - fy1
        fx1 = xs - x0.astype(jnp.float32)
        fx0 = 1.0 - fx1
        wy0 = jnp.where((y0 >= 0) & (y0 <= H - 1), fy0, 0.0)
        wy1 = jnp.where((y0 >= -1) & (y0 <= H - 2), fy1, 0.0)
        wx0 = jnp.where((x0 >= 0) & (x0 <= W - 1), fx0, 0.0)
        wx1 = jnp.where((x0 >= -1) & (x0 <= W - 2), fx1, 0.0)
        yc0 = jnp.clip(y0, 0, H - 1)
        yc1 = jnp.clip(y0 + 1, 0, H - 1)
        xc0 = jnp.clip(x0, 0, W - 1)
        xc1 = jnp.clip(x0 + 1, 0, W - 1)
        r0 = yc0 * W
        r1 = yc1 * W
        i00[pl.ds(s, L)] = r0 + xc0
        i01[pl.ds(s, L)] = r0 + xc1
        i10[pl.ds(s, L)] = r1 + xc0
        i11[pl.ds(s, L)] = r1 + xc1
        w00[pl.ds(s, L)] = wy0 * wx0
        w01[pl.ds(s, L)] = wy0 * wx1
        w10[pl.ds(s, L)] = wy1 * wx0
        w11[pl.ds(s, L)] = wy1 * wx1
        return carry

    lax.fori_loop(0, NG, ibody, None)

    plane_sets = ((pA0, pA1, pA2), (pB0, pB1, pB2))
    sem_sets = ((sA0, sA1, sA2), (sB0, sB1, sB2))
    outs = (ob0, ob1, ob2)
    row0 = b * C + cbase

    def load_block(blk, setidx):
        return [
            pltpu.async_copy(x_hbm.at[row0 + blk * NCH + t],
                             plane_sets[setidx][t], sem_sets[setidx][t])
            for t in range(NCH)
        ]

    handles = [None, None]
    handles[0] = load_block(0, 0)
    for blk in range(NBLK):
        cur = blk % 2
        nxt = 1 - cur
        if blk + 1 < NBLK:
            handles[nxt] = load_block(blk + 1, nxt)
        for h in handles[cur]:
            h.wait()
        pls = plane_sets[cur]

        def gbody(g, carry):
            s = g * L
            a00 = i00[pl.ds(s, L)]
            a01 = i01[pl.ds(s, L)]
            a10 = i10[pl.ds(s, L)]
            a11 = i11[pl.ds(s, L)]
            b00 = w00[pl.ds(s, L)]
            b01 = w01[pl.ds(s, L)]
            b10 = w10[pl.ds(s, L)]
            b11 = w11[pl.ds(s, L)]
            for t in range(NCH):
                acc = plsc.load_gather(pls[t], [a00]) * b00
                acc = acc + plsc.load_gather(pls[t], [a01]) * b01
                acc = acc + plsc.load_gather(pls[t], [a10]) * b10
                acc = acc + plsc.load_gather(pls[t], [a11]) * b11
                outs[t][pl.ds(s, L)] = acc
            return carry

        lax.fori_loop(0, NG, gbody, None)
        for t in range(NCH):
            pltpu.sync_copy(outs[t], out_hbm.at[row0 + blk * NCH + t])


def kernel(x, offset, grid_size):
    shift = jnp.asarray(grid_size).astype(jnp.float32) - jnp.float32(GH)
    off = offset.reshape(B, 2 * P) + shift
    out = _defem_sc(x.reshape(NPLANES, HW), off)
    return out.reshape(B, C, GH, GW)


# speedup vs baseline: 7.7211x; 1.4434x over previous
import functools

import jax
import jax.numpy as jnp
from jax import lax
from jax.experimental import pallas as pl
from jax.experimental.pallas import tpu as pltpu
from jax.experimental.pallas import tpu_sc as plsc

B, C, H, W = 8, 192, 112, 112
GH, GW = 56, 56
P = GH * GW
L = 16
NG = P // L
NWORK = 32
WPB = NWORK // B
CPW = C // WPB

SY = float(H) / GH
SX = float(W) / GW
BY = SY * 0.5 - 0.5
BX = SX * 0.5 - 0.5

NCH = 2
NBLK = CPW // NCH


def _ifloor(v):
    t = v.astype(jnp.int32)
    return jnp.where(t.astype(jnp.float32) > v, t - 1, t)


_mesh = plsc.VectorSubcoreMesh(core_axis_name="c", subcore_axis_name="s")


@functools.partial(
    pl.kernel,
    mesh=_mesh,
    compiler_params=pltpu.CompilerParams(needs_layout_passes=False),
    out_type=jax.ShapeDtypeStruct((B, C, GH, GW), jnp.float32),
    scratch_types=[
        pltpu.VMEM((2 * P,), jnp.float32),
        pltpu.VMEM((P,), jnp.int32),
        pltpu.VMEM((P,), jnp.int32),
        pltpu.VMEM((P,), jnp.int32),
        pltpu.VMEM((P,), jnp.int32),
        pltpu.VMEM((P,), jnp.float32),
        pltpu.VMEM((P,), jnp.float32),
        pltpu.VMEM((P,), jnp.float32),
        pltpu.VMEM((P,), jnp.float32),
        pltpu.VMEM((H, W), jnp.float32),
        pltpu.VMEM((H, W), jnp.float32),
        pltpu.VMEM((H, W), jnp.float32),
        pltpu.VMEM((H, W), jnp.float32),
        pltpu.VMEM((GH, GW), jnp.float32),
        pltpu.VMEM((GH, GW), jnp.float32),
        pltpu.SemaphoreType.DMA,
        pltpu.SemaphoreType.DMA,
        pltpu.SemaphoreType.DMA,
        pltpu.SemaphoreType.DMA,
    ],
)
def _defem_sc(x_hbm, off_hbm, out_hbm,
              off_v, y0a, y1a, x0a, x1a, w00, w01, w10, w11,
              pA0, pA1, pB0, pB1, ob0, ob1,
              sA0, sA1, sB0, sB1):
    cid = lax.axis_index("c")
    sid = lax.axis_index("s")
    wid = sid * 2 + cid
    b = lax.div(wid, jnp.int32(WPB))
    cbase = lax.rem(wid, jnp.int32(WPB)) * CPW

    pltpu.sync_copy(off_hbm.at[b], off_v)

    def ibody(g, carry):
        s = g * L
        lane = lax.iota(jnp.int32, L)
        p = s + lane
        pi = lax.div(p, jnp.int32(GW))
        pj = p - pi * GW
        ys = pi.astype(jnp.float32) * SY + BY + off_v[pl.ds(s, L)]
        xs = pj.astype(jnp.float32) * SX + BX + off_v[pl.ds(P + s, L)]
        y0 = _ifloor(ys)
        x0 = _ifloor(xs)
        fy1 = ys - y0.astype(jnp.float32)
        fy0 = 1.0 - fy1
        fx1 = xs - x0.astype(jnp.float32)
        fx0 = 1.0 - fx1
        wy0 = jnp.where((y0 >= 0) & (y0 <= H - 1), fy0, 0.0)
        wy1 = jnp.where((y0 >= -1) & (y0 <= H - 2), fy1, 0.0)
        wx0 = jnp.where((x0 >= 0) & (x0 <= W - 1), fx0, 0.0)
        wx1 = jnp.where((x0 >= -1) & (x0 <= W - 2), fx1, 0.0)
        y0a[pl.ds(s, L)] = jnp.clip(y0, 0, H - 1)
        y1a[pl.ds(s, L)] = jnp.clip(y0 + 1, 0, H - 1)
        x0a[pl.ds(s, L)] = jnp.clip(x0, 0, W - 1)
        x1a[pl.ds(s, L)] = jnp.clip(x0 + 1, 0, W - 1)
        w00[pl.ds(s, L)] = wy0 * wx0
        w01[pl.ds(s, L)] = wy0 * wx1
        w10[pl.ds(s, L)] = wy1 * wx0
        w11[pl.ds(s, L)] = wy1 * wx1
        return carry

    lax.fori_loop(0, NG, ibody, None)

    plane_sets = ((pA0, pA1), (pB0, pB1))
    sem_sets = ((sA0, sA1), (sB0, sB1))
    outs = (ob0, ob1)

    def load_block(blk, setidx):
        return [
            pltpu.async_copy(x_hbm.at[b, cbase + blk * NCH + t],
                             plane_sets[setidx][t], sem_sets[setidx][t])
            for t in range(NCH)
        ]

    handles = [None, None]
    handles[0] = load_block(0, 0)
    for blk in range(NBLK):
        cur = blk % 2
        nxt = 1 - cur
        if blk + 1 < NBLK:
            handles[nxt] = load_block(blk + 1, nxt)
        for h in handles[cur]:
            h.wait()
        pls = plane_sets[cur]

        def gbody(g, carry):
            s = g * L
            ay0 = y0a[pl.ds(s, L)]
            ay1 = y1a[pl.ds(s, L)]
            ax0 = x0a[pl.ds(s, L)]
            ax1 = x1a[pl.ds(s, L)]
            p = s + lax.iota(jnp.int32, L)
            vpi = lax.div(p, jnp.int32(GW))
            vpj = p - vpi * GW
            b00 = w00[pl.ds(s, L)]
            b01 = w01[pl.ds(s, L)]
            b10 = w10[pl.ds(s, L)]
            b11 = w11[pl.ds(s, L)]
            for t in range(NCH):
                acc = plsc.load_gather(pls[t], [ay0, ax0]) * b00
                acc = acc + plsc.load_gather(pls[t], [ay0, ax1]) * b01
                acc = acc + plsc.load_gather(pls[t], [ay1, ax0]) * b10
                acc = acc + plsc.load_gather(pls[t], [ay1, ax1]) * b11
                plsc.store_scatter(outs[t], [vpi, vpj], acc)
            return carry

        lax.fori_loop(0, NG, gbody, None)
        for t in range(NCH):
            pltpu.sync_copy(outs[t], out_hbm.at[b, cbase + blk * NCH + t])


def kernel(x, offset, grid_size):
    shift = jnp.asarray(grid_size).astype(jnp.float32) - jnp.float32(GH)
    off = offset.reshape(B, 2 * P) + shift
    return _defem_sc(x, off)
